# Initial kernel scaffold; baseline (speedup 1.0000x reference)
import jax
import jax.numpy as jnp
from jax.experimental import pallas as pl


def kernel(pois_embs, src_indices, src_values, tar_indices, tar_values):
    raise NotImplementedError("write your pallas kernel here")


# speedup vs baseline: 2.4951x; 2.4951x over previous
import jax
import jax.numpy as jnp
from jax import lax
from jax.experimental import pallas as pl
from jax.experimental.pallas import tpu as pltpu
from jax.experimental.pallas import tpu_sc as plsc

N = 10000
D = 256
DH = 128
NNZ = 160000

NC = 2
NS = 16
NPAD = 10112
NNZP = 163840
EPT = NNZP // NS
E = 320
NCHUNK = EPT // E
RPT = NPAD // NS
CH = 40
NCH = N // CH
KMAX = (NCH + NS - 1) // NS


def _body(xh, tr, tcl, tv, sr, scl, sv, zr, out,
          m, x1, x2, x3,
          colbuf, cshift, ridx, valbuf, rows_buf, acc, sem):
    c = lax.axis_index("c")
    s = lax.axis_index("s")
    coff = c * NPAD

    def spmm(rows_hbm, cols_hbm, vals_hbm, xsrc, init, dst):
        pltpu.sync_copy(init.at[pl.ds(coff + s * RPT, RPT)],
                        acc.at[pl.ds(s * RPT, RPT)])
        plsc.subcore_barrier()

        base0 = s * EPT

        def chunk_body(k, carry):
            base = base0 + k * E
            pltpu.sync_copy(cols_hbm.at[pl.ds(base, E)], colbuf)
            pltpu.sync_copy(rows_hbm.at[pl.ds(base, E)], ridx)
            pltpu.sync_copy(vals_hbm.at[pl.ds(base, E)], valbuf)

            def shift_body(i, cy):
                sl = pl.ds(i * 16, 16)
                cshift[sl] = colbuf[sl] + coff
                return cy

            lax.fori_loop(0, E // 16, shift_body, 0, unroll=2)
            pltpu.async_copy(xsrc.at[cshift], rows_buf, sem).wait()

            def scale_body(g, cy):
                vv = valbuf[pl.ds(g * 16, 16)]
                for t in range(16):
                    v = vv[t]
                    e = g * 16 + t
                    for j in range(DH // 16):
                        sl = pl.ds(j * 16, 16)
                        rows_buf[e, sl] = rows_buf[e, sl] * v
                return cy

            lax.fori_loop(0, E // 16, scale_body, 0)
            pltpu.sync_copy(rows_buf, acc.at[ridx], add=True)
            return carry

        lax.fori_loop(0, NCHUNK, chunk_body, 0)
        plsc.subcore_barrier()
        pltpu.sync_copy(acc.at[pl.ds(s * RPT, RPT)],
                        dst.at[pl.ds(coff + s * RPT, RPT)])
        plsc.subcore_barrier()

    spmm(tr, tcl, tv, xh, zr, m)
    spmm(sr, scl, sv, m, xh, x1)
    spmm(tr, tcl, tv, x1, zr, m)
    spmm(sr, scl, sv, m, x1, x2)
    spmm(tr, tcl, tv, x2, zr, m)
    spmm(sr, scl, sv, m, x2, x3)

    for k in range(KMAX):
        cid = s + k * NS

        @pl.when(cid < NCH)
        def _():
            r0 = cid * CH
            pltpu.sync_copy(xh.at[pl.ds(coff + r0, CH)], rows_buf.at[pl.ds(0, CH)])
            pltpu.sync_copy(x1.at[pl.ds(coff + r0, CH)], rows_buf.at[pl.ds(CH, CH)])
            pltpu.sync_copy(x2.at[pl.ds(coff + r0, CH)], rows_buf.at[pl.ds(2 * CH, CH)])
            pltpu.sync_copy(x3.at[pl.ds(coff + r0, CH)], rows_buf.at[pl.ds(3 * CH, CH)])

            def mean_body(r, cy):
                for j in range(DH // 16):
                    sl = pl.ds(j * 16, 16)
                    rows_buf[r, sl] = (rows_buf[r, sl] + rows_buf[CH + r, sl]
                                       + rows_buf[2 * CH + r, sl]
                                       + rows_buf[3 * CH + r, sl]) * 0.25
                return cy

            lax.fori_loop(0, CH, mean_body, 0)
            pltpu.sync_copy(rows_buf.at[pl.ds(0, CH)],
                            out.at[pl.ds(r0, CH), pl.ds(c * DH, DH)])


_mesh = plsc.VectorSubcoreMesh(core_axis_name="c", subcore_axis_name="s")

_call = pl.kernel(
    _body,
    out_type=jax.ShapeDtypeStruct((N, D), jnp.float32),
    mesh=_mesh,
    scratch_types=[
        pltpu.HBM((2 * NPAD, DH), jnp.float32),
        pltpu.HBM((2 * NPAD, DH), jnp.float32),
        pltpu.HBM((2 * NPAD, DH), jnp.float32),
        pltpu.HBM((2 * NPAD, DH), jnp.float32),
        pltpu.VMEM((E,), jnp.int32),
        pltpu.VMEM((E,), jnp.int32),
        pltpu.VMEM((E,), jnp.int32),
        pltpu.VMEM((E,), jnp.float32),
        pltpu.VMEM((E, DH), jnp.float32),
        pltpu.VMEM_SHARED((NPAD, DH), jnp.float32),
        pltpu.SemaphoreType.DMA,
    ],
)


def kernel(pois_embs, src_indices, src_values, tar_indices, tar_values):
    xh = jnp.zeros((2 * NPAD, DH), jnp.float32)
    xh = xh.at[:N].set(pois_embs[:, :DH]).at[NPAD:NPAD + N].set(pois_embs[:, DH:])
    def pad_edges(indices, values):
        rows = jnp.zeros((NNZP,), jnp.int32).at[:NNZ].set(
            indices[0].astype(jnp.int32))
        cols = jnp.zeros((NNZP,), jnp.int32).at[:NNZ].set(
            indices[1].astype(jnp.int32))
        vals = jnp.zeros((NNZP,), jnp.float32).at[:NNZ].set(values)
        return rows, cols, vals

    tr, tcl, tv = pad_edges(tar_indices, tar_values)
    sr, scl, sv = pad_edges(src_indices, src_values)
    zr = jnp.zeros((2 * NPAD, DH), jnp.float32)
    return _call(xh, tr, tcl, tv, sr, scl, sv, zr)
